# baseline (speedup 1.0000x reference)
import functools

import jax
import jax.numpy as jnp
import numpy as np
from jax.experimental import pallas as pl
from jax.experimental.pallas import tpu as pltpu
from jax.experimental.pallas import tpu_sc as plsc

IMAGE_SIZE = 128
EPS = 1e-8
_F32 = np.float32
_BIG = np.float32(1e30)
_TILES = 4
_ROWS = 8 * _TILES
_UNROLL = 2


def _raster_kernel(f_total, f_padded, *refs):
    (cs_r, o_p2f, o_z, o_b0, o_b1, o_b2, o_d) = refs
    W = IMAGE_SIZE
    b = pl.program_id(0)
    hblk = pl.program_id(1)

    ix = jax.lax.broadcasted_iota(jnp.int32, (8, W), 1).astype(_F32)
    pxb = 1.0 - (2.0 * ix + 1.0) * _F32(1.0 / W)
    iy = jax.lax.broadcasted_iota(jnp.int32, (8, W), 0)
    base = hblk * _ROWS
    pybs = []
    for t in range(_TILES):
        yt = (base + t * 8 + iy).astype(_F32)
        pybs.append(_F32(1.0) - (2.0 * yt + 1.0) * _F32(1.0 / IMAGE_SIZE))

    zeros = jnp.zeros((8, W), _F32)

    st0 = []
    for t in range(_TILES):
        st0.extend([jnp.full((8, W), _BIG, _F32), zeros, zeros, zeros,
                    zeros, zeros, zeros])

    def body(i, st):
        del i
        st = list(st)
        fbase = st[-1]
        for k in range(_UNROLL):
            f = jax.lax.add(fbase, np.int32(k))
            n0x = cs_r[0, 0, f]
            n0y = cs_r[0, 1, f]
            c0 = cs_r[0, 2, f]
            n1x = cs_r[0, 3, f]
            n1y = cs_r[0, 4, f]
            c1 = cs_r[0, 5, f]
            zx = cs_r[0, 6, f]
            zy = cs_r[0, 7, f]
            zc = cs_r[0, 8, f]
            q0 = cs_r[0, 9, f]
            q1 = cs_r[0, 10, f]
            q2 = cs_r[0, 11, f]
            ff = f.astype(_F32)
            for t in range(_TILES):
                sti = 7 * t
                zrun, fidx, pb0, pb1, pq0, pq1, pq2 = st[sti: sti + 7]
                pyb = pybs[t]
                b0 = n0x * pxb + (n0y * pyb + c0)
                b1 = n1x * pxb + (n1y * pyb + c1)
                b2 = 1.0 - b0 - b1
                pz = zx * pxb + (zy * pyb + zc)
                m3 = jnp.minimum(jnp.minimum(b0, b1), b2)
                zcand = jnp.where(m3 >= 0.0, pz, _BIG)
                upd = zcand < zrun
                st[sti: sti + 7] = [
                    jnp.minimum(zcand, zrun),
                    jnp.where(upd, ff, fidx),
                    jnp.where(upd, b0, pb0),
                    jnp.where(upd, b1, pb1),
                    jnp.where(upd, q0, pq0),
                    jnp.where(upd, q1, pq1),
                    jnp.where(upd, q2, pq2),
                ]
        st[-1] = jax.lax.add(fbase, np.int32(_UNROLL))
        return tuple(st)

    st0.append(jnp.int32(0))
    st = jax.lax.fori_loop(np.int32(0), np.int32(f_padded // _UNROLL),
                           body, tuple(st0))

    bF = (b * f_total).astype(_F32)
    rows_p2f, rows_z, rows_b0, rows_b1, rows_b2, rows_d = [], [], [], [], [], []
    for t in range(_TILES):
        zrun, fidx, pb0, pb1, pq0, pq1, pq2 = st[7 * t: 7 * t + 7]
        pb2 = 1.0 - pb0 - pb1
        hit = zrun < _BIG
        d = jnp.minimum(jnp.minimum(pb0 * pb0 * pq0, pb1 * pb1 * pq1),
                        pb2 * pb2 * pq2)
        rows_p2f.append(jnp.where(hit, bF + fidx, _F32(-1.0)))
        rows_z.append(jnp.where(hit, zrun, _F32(-1.0)))
        rows_b0.append(jnp.where(hit, pb0, _F32(-1.0)))
        rows_b1.append(jnp.where(hit, pb1, _F32(-1.0)))
        rows_b2.append(jnp.where(hit, pb2, _F32(-1.0)))
        rows_d.append(jnp.where(hit, -d, _F32(-1.0)))
    o_p2f[0] = jnp.concatenate(rows_p2f, axis=0).astype(jnp.int32)
    o_z[0] = jnp.concatenate(rows_z, axis=0)
    o_b0[0] = jnp.concatenate(rows_b0, axis=0)
    o_b1[0] = jnp.concatenate(rows_b1, axis=0)
    o_b2[0] = jnp.concatenate(rows_b2, axis=0)
    o_d[0] = jnp.concatenate(rows_d, axis=0)


_SC_NC = 2
_SC_NS = 16
_SC_L = 16
_NCST = 12


def _sc_face_constants(verts, faces_i, fp):
    B, V, _ = verts.shape
    F = faces_i.shape[0]
    nw = _SC_NC * _SC_NS
    chunk = nw * _SC_L
    fp3 = ((max(F, fp) + chunk - 1) // chunk) * chunk
    per_w = fp3 // nw
    jn = per_w // _SC_L

    vx = verts[:, :, 0].reshape(B * V)
    vy = verts[:, :, 1].reshape(B * V)
    vz = verts[:, :, 2].reshape(B * V)
    f0 = jnp.pad(faces_i[:, 0], (0, fp3 - F))
    f1 = jnp.pad(faces_i[:, 1], (0, fp3 - F))
    f2 = jnp.pad(faces_i[:, 2], (0, fp3 - F))

    mesh = plsc.VectorSubcoreMesh(core_axis_name="c", subcore_axis_name="s")

    @functools.partial(
        pl.kernel, mesh=mesh,
        out_type=jax.ShapeDtypeStruct((B * _NCST * fp3,), jnp.float32),
        scratch_types=[
            pltpu.VMEM((per_w,), jnp.int32),
            pltpu.VMEM((per_w,), jnp.int32),
            pltpu.VMEM((per_w,), jnp.int32),
            pltpu.VMEM((per_w,), jnp.int32),
            pltpu.VMEM((per_w,), jnp.int32),
            pltpu.VMEM((per_w,), jnp.int32),
            pltpu.VMEM((9 * per_w,), jnp.float32),
            pltpu.VMEM((_NCST * per_w,), jnp.float32),
            pltpu.SemaphoreType.DMA,
        ],
    )
    def sck(vx_h, vy_h, vz_h, f0_h, f1_h, f2_h, out_h,
            f0v, f1v, f2v, i0v, i1v, i2v, gv, outv, sem):
        c = jax.lax.axis_index("c")
        s = jax.lax.axis_index("s")
        wid = jax.lax.add(jax.lax.mul(s, np.int32(_SC_NC)), c)
        base = jax.lax.mul(wid, np.int32(per_w))
        pltpu.sync_copy(f0_h.at[pl.ds(base, per_w)], f0v)
        pltpu.sync_copy(f1_h.at[pl.ds(base, per_w)], f1v)
        pltpu.sync_copy(f2_h.at[pl.ds(base, per_w)], f2v)
        onev = jnp.full((_SC_L,), _F32(1.0), jnp.float32)
        epsv = jnp.full((_SC_L,), _F32(EPS), jnp.float32)
        negv = jnp.full((_SC_L,), _F32(-1.0), jnp.float32)
        zerov = jnp.zeros((_SC_L,), jnp.float32)
        for b in range(B):
            boff = jnp.full((_SC_L,), b * V, jnp.int32)
            for j in range(jn):
                sl = pl.ds(j * _SC_L, _SC_L)
                i0v[sl] = f0v[sl] + boff
                i1v[sl] = f1v[sl] + boff
                i2v[sl] = f2v[sl] + boff
            copies = []
            for iv, row in ((i0v, 0), (i1v, 1), (i2v, 2)):
                for coord, src in enumerate((vx_h, vy_h, vz_h)):
                    dst = gv.at[pl.ds((row * 3 + coord) * per_w, per_w)]
                    copies.append(pltpu.async_copy(src.at[iv], dst, sem))
            for cp in copies:
                cp.wait()
            for j in range(jn):
                sl = pl.ds(j * _SC_L, _SC_L)
                def gld(row):
                    return gv[pl.ds(row * per_w + j * _SC_L, _SC_L)]

                x0 = gld(0)
                y0 = gld(1)
                z0 = gld(2)
                x1 = gld(3)
                y1 = gld(4)
                z1 = gld(5)
                x2 = gld(6)
                y2 = gld(7)
                z2 = gld(8)
                area = (x1 - x0) * (y2 - y0) - (y1 - y0) * (x2 - x0)
                absa = jnp.abs(area)
                valid = absa > epsv
                asafe = jnp.where(absa < epsv, epsv, area)
                inv = onev / asafe
                n0x = -(y2 - y1) * inv
                n0y = (x2 - x1) * inv
                c0 = ((y2 - y1) * x1 - (x2 - x1) * y1) * inv
                n1x = -(y0 - y2) * inv
                n1y = (x0 - x2) * inv
                c1 = ((y0 - y2) * x2 - (x0 - x2) * y2) * inv
                n2x = -(y1 - y0) * inv
                n2y = (x1 - x0) * inv
                c2 = ((y1 - y0) * x0 - (x1 - x0) * y0) * inv
                n0x = jnp.where(valid, n0x, zerov)
                n0y = jnp.where(valid, n0y, zerov)
                c0 = jnp.where(valid, c0, negv)
                zx = n0x * z0 + n1x * z1 + n2x * z2
                zy = n0y * z0 + n1y * z1 + n2y * z2
                zc = c0 * z0 + c1 * z1 + c2 * z2
                area2 = area * area
                e0x = x2 - x1
                e0y = y2 - y1
                e1x = x0 - x2
                e1y = y0 - y2
                e2x = x1 - x0
                e2y = y1 - y0
                q0 = jnp.where(valid, area2 / (e0x * e0x + e0y * e0y), zerov)
                q1 = jnp.where(valid, area2 / (e1x * e1x + e1y * e1y), zerov)
                q2 = jnp.where(valid, area2 / (e2x * e2x + e2y * e2y), zerov)
                vals = (n0x, n0y, c0, n1x, n1y, c1, zx, zy, zc, q0, q1, q2)
                for k, v in enumerate(vals):
                    outv[pl.ds(k * per_w + j * _SC_L, _SC_L)] = v
            for k in range(_NCST):
                off = jax.lax.add(base, np.int32((b * _NCST + k) * fp3))
                pltpu.sync_copy(outv.at[pl.ds(k * per_w, per_w)],
                                out_h.at[pl.ds(off, per_w)])

    out = sck(vx, vy, vz, f0, f1, f2)
    return out.reshape(B, _NCST, fp3)


def _face_constants(verts, faces):
    fv = verts[:, faces]
    x0 = fv[..., 0, 0]
    y0 = fv[..., 0, 1]
    z0 = fv[..., 0, 2]
    x1 = fv[..., 1, 0]
    y1 = fv[..., 1, 1]
    z1 = fv[..., 1, 2]
    x2 = fv[..., 2, 0]
    y2 = fv[..., 2, 1]
    z2 = fv[..., 2, 2]
    area = (x1 - x0) * (y2 - y0) - (y1 - y0) * (x2 - x0)
    valid = jnp.abs(area) > EPS
    asafe = jnp.where(jnp.abs(area) < EPS, _F32(EPS), area)
    inv = _F32(1.0) / asafe
    n0x = -(y2 - y1) * inv
    n0y = (x2 - x1) * inv
    c0 = ((y2 - y1) * x1 - (x2 - x1) * y1) * inv
    n1x = -(y0 - y2) * inv
    n1y = (x0 - x2) * inv
    c1 = ((y0 - y2) * x2 - (x0 - x2) * y2) * inv
    n2x = -(y1 - y0) * inv
    n2y = (x1 - x0) * inv
    c2 = ((y1 - y0) * x0 - (x1 - x0) * y0) * inv
    n0x = jnp.where(valid, n0x, 0.0)
    n0y = jnp.where(valid, n0y, 0.0)
    c0 = jnp.where(valid, c0, -1.0)
    zx = n0x * z0 + n1x * z1 + n2x * z2
    zy = n0y * z0 + n1y * z1 + n2y * z2
    zc = c0 * z0 + c1 * z1 + c2 * z2
    return (n0x, n0y, c0, n1x, n1y, c1,
            zx, zy, zc, x0, y0, x1, y1, x2, y2)


@jax.jit
def _run(verts, faces):
    B = verts.shape[0]
    F = faces.shape[0]
    H = W = IMAGE_SIZE
    faces_i = faces.astype(jnp.int32)
    fpad = ((F + _UNROLL - 1) // _UNROLL) * _UNROLL
    cst = _sc_face_constants(verts.astype(_F32), faces_i, fpad)
    fp3 = cst.shape[2]

    z32 = lambda: jnp.int32(0)
    cspec = pl.BlockSpec((1, _NCST, fp3), lambda b, h: (b, z32(), z32()),
                         memory_space=pltpu.SMEM)
    ospec = pl.BlockSpec((1, _ROWS, W), lambda b, h: (b, h, z32()))
    outs = pl.pallas_call(
        functools.partial(_raster_kernel, F, fpad),
        grid=(B, H // _ROWS),
        in_specs=[cspec],
        out_specs=[ospec] * 6,
        out_shape=[
            jax.ShapeDtypeStruct((B, H, W), jnp.int32),
            jax.ShapeDtypeStruct((B, H, W), _F32),
            jax.ShapeDtypeStruct((B, H, W), _F32),
            jax.ShapeDtypeStruct((B, H, W), _F32),
            jax.ShapeDtypeStruct((B, H, W), _F32),
            jax.ShapeDtypeStruct((B, H, W), _F32),
        ],
    )(cst)
    p2f_i, zb, b0, b1, b2, ds = outs
    pix_to_face = p2f_i.astype(jnp.int64)[..., None]
    zbuf = zb[..., None]
    bary = jnp.stack([b0, b1, b2], axis=-1)[:, :, :, None, :]
    dists = ds[..., None]
    return pix_to_face, zbuf, bary, dists


def kernel(verts, faces):
    return _run(verts, faces)


# speedup vs baseline: 1.0726x; 1.0726x over previous
import functools

import jax
import jax.numpy as jnp
import numpy as np
from jax.experimental import pallas as pl
from jax.experimental.pallas import tpu as pltpu
from jax.experimental.pallas import tpu_sc as plsc

IMAGE_SIZE = 128
EPS = 1e-8
_F32 = np.float32
_BIG = np.float32(1e30)
_TILES = 4
_ROWS = 8 * _TILES
_UNROLL = 2


def _raster_kernel(f_total, f_padded, *refs):
    (n0x_r, n0y_r, c0_r, n1x_r, n1y_r, c1_r,
     zx_r, zy_r, zc_r, q0_r, q1_r, q2_r,
     o_p2f, o_z, o_b0, o_b1, o_b2, o_d) = refs
    W = IMAGE_SIZE
    b = pl.program_id(0)
    hblk = pl.program_id(1)

    ix = jax.lax.broadcasted_iota(jnp.int32, (8, W), 1).astype(_F32)
    pxb = 1.0 - (2.0 * ix + 1.0) * _F32(1.0 / W)
    iy = jax.lax.broadcasted_iota(jnp.int32, (8, W), 0)
    base = hblk * _ROWS
    pybs = []
    for t in range(_TILES):
        yt = (base + t * 8 + iy).astype(_F32)
        pybs.append(_F32(1.0) - (2.0 * yt + 1.0) * _F32(1.0 / IMAGE_SIZE))

    zeros = jnp.zeros((8, W), _F32)

    st0 = []
    for t in range(_TILES):
        st0.extend([jnp.full((8, W), _BIG, _F32), zeros, zeros, zeros,
                    zeros, zeros, zeros])

    def body(i, st):
        del i
        st = list(st)
        fbase = st[-1]
        for k in range(_UNROLL):
            f = jax.lax.add(fbase, np.int32(k))
            n0x = n0x_r[0, 0, f]
            n0y = n0y_r[0, 0, f]
            c0 = c0_r[0, 0, f]
            n1x = n1x_r[0, 0, f]
            n1y = n1y_r[0, 0, f]
            c1 = c1_r[0, 0, f]
            zx = zx_r[0, 0, f]
            zy = zy_r[0, 0, f]
            zc = zc_r[0, 0, f]
            q0 = q0_r[0, 0, f]
            q1 = q1_r[0, 0, f]
            q2 = q2_r[0, 0, f]
            ff = f.astype(_F32)
            for t in range(_TILES):
                sti = 7 * t
                zrun, fidx, pb0, pb1, pq0, pq1, pq2 = st[sti: sti + 7]
                pyb = pybs[t]
                b0 = n0x * pxb + (n0y * pyb + c0)
                b1 = n1x * pxb + (n1y * pyb + c1)
                b2 = 1.0 - b0 - b1
                pz = zx * pxb + (zy * pyb + zc)
                m3 = jnp.minimum(jnp.minimum(b0, b1), b2)
                zcand = jnp.where(m3 >= 0.0, pz, _BIG)
                upd = zcand < zrun
                st[sti: sti + 7] = [
                    jnp.minimum(zcand, zrun),
                    jnp.where(upd, ff, fidx),
                    jnp.where(upd, b0, pb0),
                    jnp.where(upd, b1, pb1),
                    jnp.where(upd, q0, pq0),
                    jnp.where(upd, q1, pq1),
                    jnp.where(upd, q2, pq2),
                ]
        st[-1] = jax.lax.add(fbase, np.int32(_UNROLL))
        return tuple(st)

    st0.append(jnp.int32(0))
    st = jax.lax.fori_loop(np.int32(0), np.int32(f_padded // _UNROLL),
                           body, tuple(st0))

    bF = (b * f_total).astype(_F32)
    rows_p2f, rows_z, rows_b0, rows_b1, rows_b2, rows_d = [], [], [], [], [], []
    for t in range(_TILES):
        zrun, fidx, pb0, pb1, pq0, pq1, pq2 = st[7 * t: 7 * t + 7]
        pb2 = 1.0 - pb0 - pb1
        hit = zrun < _BIG
        d = jnp.minimum(jnp.minimum(pb0 * pb0 * pq0, pb1 * pb1 * pq1),
                        pb2 * pb2 * pq2)
        rows_p2f.append(jnp.where(hit, bF + fidx, _F32(-1.0)))
        rows_z.append(jnp.where(hit, zrun, _F32(-1.0)))
        rows_b0.append(jnp.where(hit, pb0, _F32(-1.0)))
        rows_b1.append(jnp.where(hit, pb1, _F32(-1.0)))
        rows_b2.append(jnp.where(hit, pb2, _F32(-1.0)))
        rows_d.append(jnp.where(hit, -d, _F32(-1.0)))
    o_p2f[0] = jnp.concatenate(rows_p2f, axis=0).astype(jnp.int32)
    o_z[0] = jnp.concatenate(rows_z, axis=0)
    o_b0[0] = jnp.concatenate(rows_b0, axis=0)
    o_b1[0] = jnp.concatenate(rows_b1, axis=0)
    o_b2[0] = jnp.concatenate(rows_b2, axis=0)
    o_d[0] = jnp.concatenate(rows_d, axis=0)


_SC_NC = 2
_SC_NS = 16
_SC_L = 16
_NCST = 12


def _sc_face_constants(verts, faces_i, fp):
    B, V, _ = verts.shape
    F = faces_i.shape[0]
    nw = _SC_NC * _SC_NS
    chunk = nw * _SC_L
    fp3 = ((max(F, fp) + chunk - 1) // chunk) * chunk
    per_w = fp3 // nw
    jn = per_w // _SC_L

    vx = verts[:, :, 0].reshape(B * V)
    vy = verts[:, :, 1].reshape(B * V)
    vz = verts[:, :, 2].reshape(B * V)
    f0 = jnp.pad(faces_i[:, 0], (0, fp3 - F))
    f1 = jnp.pad(faces_i[:, 1], (0, fp3 - F))
    f2 = jnp.pad(faces_i[:, 2], (0, fp3 - F))

    mesh = plsc.VectorSubcoreMesh(core_axis_name="c", subcore_axis_name="s")

    @functools.partial(
        pl.kernel, mesh=mesh,
        out_type=jax.ShapeDtypeStruct((B * _NCST * fp3,), jnp.float32),
        scratch_types=[
            pltpu.VMEM((per_w,), jnp.int32),
            pltpu.VMEM((per_w,), jnp.int32),
            pltpu.VMEM((per_w,), jnp.int32),
            pltpu.VMEM((per_w,), jnp.int32),
            pltpu.VMEM((per_w,), jnp.int32),
            pltpu.VMEM((per_w,), jnp.int32),
            pltpu.VMEM((9 * per_w,), jnp.float32),
            pltpu.VMEM((_NCST * per_w,), jnp.float32),
            pltpu.SemaphoreType.DMA,
        ],
    )
    def sck(vx_h, vy_h, vz_h, f0_h, f1_h, f2_h, out_h,
            f0v, f1v, f2v, i0v, i1v, i2v, gv, outv, sem):
        c = jax.lax.axis_index("c")
        s = jax.lax.axis_index("s")
        wid = jax.lax.add(jax.lax.mul(s, np.int32(_SC_NC)), c)
        base = jax.lax.mul(wid, np.int32(per_w))
        pltpu.sync_copy(f0_h.at[pl.ds(base, per_w)], f0v)
        pltpu.sync_copy(f1_h.at[pl.ds(base, per_w)], f1v)
        pltpu.sync_copy(f2_h.at[pl.ds(base, per_w)], f2v)
        onev = jnp.full((_SC_L,), _F32(1.0), jnp.float32)
        epsv = jnp.full((_SC_L,), _F32(EPS), jnp.float32)
        negv = jnp.full((_SC_L,), _F32(-1.0), jnp.float32)
        zerov = jnp.zeros((_SC_L,), jnp.float32)
        for b in range(B):
            boff = jnp.full((_SC_L,), b * V, jnp.int32)
            for j in range(jn):
                sl = pl.ds(j * _SC_L, _SC_L)
                i0v[sl] = f0v[sl] + boff
                i1v[sl] = f1v[sl] + boff
                i2v[sl] = f2v[sl] + boff
            copies = []
            for iv, row in ((i0v, 0), (i1v, 1), (i2v, 2)):
                for coord, src in enumerate((vx_h, vy_h, vz_h)):
                    dst = gv.at[pl.ds((row * 3 + coord) * per_w, per_w)]
                    copies.append(pltpu.async_copy(src.at[iv], dst, sem))
            for cp in copies:
                cp.wait()
            for j in range(jn):
                sl = pl.ds(j * _SC_L, _SC_L)
                def gld(row):
                    return gv[pl.ds(row * per_w + j * _SC_L, _SC_L)]

                x0 = gld(0)
                y0 = gld(1)
                z0 = gld(2)
                x1 = gld(3)
                y1 = gld(4)
                z1 = gld(5)
                x2 = gld(6)
                y2 = gld(7)
                z2 = gld(8)
                area = (x1 - x0) * (y2 - y0) - (y1 - y0) * (x2 - x0)
                absa = jnp.abs(area)
                valid = absa > epsv
                asafe = jnp.where(absa < epsv, epsv, area)
                inv = onev / asafe
                n0x = -(y2 - y1) * inv
                n0y = (x2 - x1) * inv
                c0 = ((y2 - y1) * x1 - (x2 - x1) * y1) * inv
                n1x = -(y0 - y2) * inv
                n1y = (x0 - x2) * inv
                c1 = ((y0 - y2) * x2 - (x0 - x2) * y2) * inv
                n2x = -(y1 - y0) * inv
                n2y = (x1 - x0) * inv
                c2 = ((y1 - y0) * x0 - (x1 - x0) * y0) * inv
                n0x = jnp.where(valid, n0x, zerov)
                n0y = jnp.where(valid, n0y, zerov)
                c0 = jnp.where(valid, c0, negv)
                zx = n0x * z0 + n1x * z1 + n2x * z2
                zy = n0y * z0 + n1y * z1 + n2y * z2
                zc = c0 * z0 + c1 * z1 + c2 * z2
                area2 = area * area
                e0x = x2 - x1
                e0y = y2 - y1
                e1x = x0 - x2
                e1y = y0 - y2
                e2x = x1 - x0
                e2y = y1 - y0
                q0 = jnp.where(valid, area2 / (e0x * e0x + e0y * e0y), zerov)
                q1 = jnp.where(valid, area2 / (e1x * e1x + e1y * e1y), zerov)
                q2 = jnp.where(valid, area2 / (e2x * e2x + e2y * e2y), zerov)
                vals = (n0x, n0y, c0, n1x, n1y, c1, zx, zy, zc, q0, q1, q2)
                for k, v in enumerate(vals):
                    outv[pl.ds(k * per_w + j * _SC_L, _SC_L)] = v
            for k in range(_NCST):
                off = jax.lax.add(base, np.int32((b * _NCST + k) * fp3))
                pltpu.sync_copy(outv.at[pl.ds(k * per_w, per_w)],
                                out_h.at[pl.ds(off, per_w)])

    out = sck(vx, vy, vz, f0, f1, f2)
    return out.reshape(B, _NCST, fp3)


@jax.jit
def _run(verts, faces):
    B = verts.shape[0]
    F = faces.shape[0]
    H = W = IMAGE_SIZE
    faces_i = faces.astype(jnp.int32)
    fpad = ((F + _UNROLL - 1) // _UNROLL) * _UNROLL
    cst = _sc_face_constants(verts.astype(_F32), faces_i, fpad)
    fp3 = cst.shape[2]
    consts = [cst[:, k:k + 1, :] for k in range(_NCST)]

    z32 = lambda: jnp.int32(0)
    cspec = pl.BlockSpec((1, 1, fp3), lambda b, h: (b, z32(), z32()),
                         memory_space=pltpu.SMEM)
    ospec = pl.BlockSpec((1, _ROWS, W), lambda b, h: (b, h, z32()))
    outs = pl.pallas_call(
        functools.partial(_raster_kernel, F, fpad),
        grid=(B, H // _ROWS),
        in_specs=[cspec] * _NCST,
        out_specs=[ospec] * 6,
        out_shape=[
            jax.ShapeDtypeStruct((B, H, W), jnp.int32),
            jax.ShapeDtypeStruct((B, H, W), _F32),
            jax.ShapeDtypeStruct((B, H, W), _F32),
            jax.ShapeDtypeStruct((B, H, W), _F32),
            jax.ShapeDtypeStruct((B, H, W), _F32),
            jax.ShapeDtypeStruct((B, H, W), _F32),
        ],
    )(*consts)
    p2f_i, zb, b0, b1, b2, ds = outs
    pix_to_face = p2f_i.astype(jnp.int64)[..., None]
    zbuf = zb[..., None]
    bary = jnp.stack([b0, b1, b2], axis=-1)[:, :, :, None, :]
    dists = ds[..., None]
    return pix_to_face, zbuf, bary, dists


def kernel(verts, faces):
    return _run(verts, faces)


# speedup vs baseline: 1.2317x; 1.1482x over previous
import functools

import jax
import jax.numpy as jnp
import numpy as np
from jax.experimental import pallas as pl
from jax.experimental.pallas import tpu as pltpu
from jax.experimental.pallas import tpu_sc as plsc

IMAGE_SIZE = 128
EPS = 1e-8
_F32 = np.float32
_BIG = np.float32(1e30)
_TILES = 4
_ROWS = 8 * _TILES
_UNROLL = 4


def _raster_kernel(f_total, f_padded, *refs):
    (n0x_r, n0y_r, c0_r, n1x_r, n1y_r, c1_r,
     zx_r, zy_r, zc_r, q0_r, q1_r, q2_r,
     o_p2f, o_z, o_b0, o_b1, o_b2, o_d) = refs
    W = IMAGE_SIZE
    b = pl.program_id(0)
    hblk = pl.program_id(1)

    ix = jax.lax.broadcasted_iota(jnp.int32, (8, W), 1).astype(_F32)
    pxb = 1.0 - (2.0 * ix + 1.0) * _F32(1.0 / W)
    iy = jax.lax.broadcasted_iota(jnp.int32, (8, W), 0)
    base = hblk * _ROWS
    pybs = []
    for t in range(_TILES):
        yt = (base + t * 8 + iy).astype(_F32)
        pybs.append(_F32(1.0) - (2.0 * yt + 1.0) * _F32(1.0 / IMAGE_SIZE))

    zeros = jnp.zeros((8, W), _F32)

    st0 = []
    for t in range(_TILES):
        st0.extend([jnp.full((8, W), _BIG, _F32), zeros, zeros, zeros,
                    zeros, zeros, zeros])

    def body(i, st):
        del i
        st = list(st)
        fbase = st[-1]
        for k in range(_UNROLL):
            f = jax.lax.add(fbase, np.int32(k))
            n0x = n0x_r[0, 0, f]
            n0y = n0y_r[0, 0, f]
            c0 = c0_r[0, 0, f]
            n1x = n1x_r[0, 0, f]
            n1y = n1y_r[0, 0, f]
            c1 = c1_r[0, 0, f]
            zx = zx_r[0, 0, f]
            zy = zy_r[0, 0, f]
            zc = zc_r[0, 0, f]
            q0 = q0_r[0, 0, f]
            q1 = q1_r[0, 0, f]
            q2 = q2_r[0, 0, f]
            ff = f.astype(_F32)
            for t in range(_TILES):
                sti = 7 * t
                zrun, fidx, pb0, pb1, pq0, pq1, pq2 = st[sti: sti + 7]
                pyb = pybs[t]
                b0 = n0x * pxb + (n0y * pyb + c0)
                b1 = n1x * pxb + (n1y * pyb + c1)
                b2 = 1.0 - b0 - b1
                pz = zx * pxb + (zy * pyb + zc)
                m3 = jnp.minimum(jnp.minimum(b0, b1), b2)
                zcand = jnp.where(m3 >= 0.0, pz, _BIG)
                upd = zcand < zrun
                st[sti: sti + 7] = [
                    jnp.minimum(zcand, zrun),
                    jnp.where(upd, ff, fidx),
                    jnp.where(upd, b0, pb0),
                    jnp.where(upd, b1, pb1),
                    jnp.where(upd, q0, pq0),
                    jnp.where(upd, q1, pq1),
                    jnp.where(upd, q2, pq2),
                ]
        st[-1] = jax.lax.add(fbase, np.int32(_UNROLL))
        return tuple(st)

    st0.append(jnp.int32(0))
    st = jax.lax.fori_loop(np.int32(0), np.int32(f_padded // _UNROLL),
                           body, tuple(st0))

    bF = (b * f_total).astype(_F32)
    rows_p2f, rows_z, rows_b0, rows_b1, rows_b2, rows_d = [], [], [], [], [], []
    for t in range(_TILES):
        zrun, fidx, pb0, pb1, pq0, pq1, pq2 = st[7 * t: 7 * t + 7]
        pb2 = 1.0 - pb0 - pb1
        hit = zrun < _BIG
        d = jnp.minimum(jnp.minimum(pb0 * pb0 * pq0, pb1 * pb1 * pq1),
                        pb2 * pb2 * pq2)
        rows_p2f.append(jnp.where(hit, bF + fidx, _F32(-1.0)))
        rows_z.append(jnp.where(hit, zrun, _F32(-1.0)))
        rows_b0.append(jnp.where(hit, pb0, _F32(-1.0)))
        rows_b1.append(jnp.where(hit, pb1, _F32(-1.0)))
        rows_b2.append(jnp.where(hit, pb2, _F32(-1.0)))
        rows_d.append(jnp.where(hit, -d, _F32(-1.0)))
    o_p2f[0] = jnp.concatenate(rows_p2f, axis=0).astype(jnp.int32)
    o_z[0] = jnp.concatenate(rows_z, axis=0)
    o_b0[0] = jnp.concatenate(rows_b0, axis=0)
    o_b1[0] = jnp.concatenate(rows_b1, axis=0)
    o_b2[0] = jnp.concatenate(rows_b2, axis=0)
    o_d[0] = jnp.concatenate(rows_d, axis=0)


_SC_NC = 2
_SC_NS = 16
_SC_L = 16
_NCST = 12


def _sc_face_constants(verts, faces_i, fp):
    B, V, _ = verts.shape
    F = faces_i.shape[0]
    nw = _SC_NC * _SC_NS
    chunk = nw * _SC_L
    fp3 = ((max(F, fp) + chunk - 1) // chunk) * chunk
    per_w = fp3 // nw
    jn = per_w // _SC_L

    vx = verts[:, :, 0].reshape(B * V)
    vy = verts[:, :, 1].reshape(B * V)
    vz = verts[:, :, 2].reshape(B * V)
    f0 = jnp.pad(faces_i[:, 0], (0, fp3 - F))
    f1 = jnp.pad(faces_i[:, 1], (0, fp3 - F))
    f2 = jnp.pad(faces_i[:, 2], (0, fp3 - F))

    mesh = plsc.VectorSubcoreMesh(core_axis_name="c", subcore_axis_name="s")

    @functools.partial(
        pl.kernel, mesh=mesh,
        out_type=jax.ShapeDtypeStruct((B * _NCST * fp3,), jnp.float32),
        scratch_types=[
            pltpu.VMEM((per_w,), jnp.int32),
            pltpu.VMEM((per_w,), jnp.int32),
            pltpu.VMEM((per_w,), jnp.int32),
            pltpu.VMEM((per_w,), jnp.int32),
            pltpu.VMEM((per_w,), jnp.int32),
            pltpu.VMEM((per_w,), jnp.int32),
            pltpu.VMEM((9 * per_w,), jnp.float32),
            pltpu.VMEM((_NCST * per_w,), jnp.float32),
            pltpu.SemaphoreType.DMA,
        ],
    )
    def sck(vx_h, vy_h, vz_h, f0_h, f1_h, f2_h, out_h,
            f0v, f1v, f2v, i0v, i1v, i2v, gv, outv, sem):
        c = jax.lax.axis_index("c")
        s = jax.lax.axis_index("s")
        wid = jax.lax.add(jax.lax.mul(s, np.int32(_SC_NC)), c)
        base = jax.lax.mul(wid, np.int32(per_w))
        pltpu.sync_copy(f0_h.at[pl.ds(base, per_w)], f0v)
        pltpu.sync_copy(f1_h.at[pl.ds(base, per_w)], f1v)
        pltpu.sync_copy(f2_h.at[pl.ds(base, per_w)], f2v)
        onev = jnp.full((_SC_L,), _F32(1.0), jnp.float32)
        epsv = jnp.full((_SC_L,), _F32(EPS), jnp.float32)
        negv = jnp.full((_SC_L,), _F32(-1.0), jnp.float32)
        zerov = jnp.zeros((_SC_L,), jnp.float32)
        for b in range(B):
            boff = jnp.full((_SC_L,), b * V, jnp.int32)
            for j in range(jn):
                sl = pl.ds(j * _SC_L, _SC_L)
                i0v[sl] = f0v[sl] + boff
                i1v[sl] = f1v[sl] + boff
                i2v[sl] = f2v[sl] + boff
            copies = []
            for iv, row in ((i0v, 0), (i1v, 1), (i2v, 2)):
                for coord, src in enumerate((vx_h, vy_h, vz_h)):
                    dst = gv.at[pl.ds((row * 3 + coord) * per_w, per_w)]
                    copies.append(pltpu.async_copy(src.at[iv], dst, sem))
            for cp in copies:
                cp.wait()
            for j in range(jn):
                sl = pl.ds(j * _SC_L, _SC_L)
                def gld(row):
                    return gv[pl.ds(row * per_w + j * _SC_L, _SC_L)]

                x0 = gld(0)
                y0 = gld(1)
                z0 = gld(2)
                x1 = gld(3)
                y1 = gld(4)
                z1 = gld(5)
                x2 = gld(6)
                y2 = gld(7)
                z2 = gld(8)
                area = (x1 - x0) * (y2 - y0) - (y1 - y0) * (x2 - x0)
                absa = jnp.abs(area)
                valid = absa > epsv
                asafe = jnp.where(absa < epsv, epsv, area)
                inv = onev / asafe
                n0x = -(y2 - y1) * inv
                n0y = (x2 - x1) * inv
                c0 = ((y2 - y1) * x1 - (x2 - x1) * y1) * inv
                n1x = -(y0 - y2) * inv
                n1y = (x0 - x2) * inv
                c1 = ((y0 - y2) * x2 - (x0 - x2) * y2) * inv
                n2x = -(y1 - y0) * inv
                n2y = (x1 - x0) * inv
                c2 = ((y1 - y0) * x0 - (x1 - x0) * y0) * inv
                n0x = jnp.where(valid, n0x, zerov)
                n0y = jnp.where(valid, n0y, zerov)
                c0 = jnp.where(valid, c0, negv)
                zx = n0x * z0 + n1x * z1 + n2x * z2
                zy = n0y * z0 + n1y * z1 + n2y * z2
                zc = c0 * z0 + c1 * z1 + c2 * z2
                area2 = area * area
                e0x = x2 - x1
                e0y = y2 - y1
                e1x = x0 - x2
                e1y = y0 - y2
                e2x = x1 - x0
                e2y = y1 - y0
                q0 = jnp.where(valid, area2 / (e0x * e0x + e0y * e0y), zerov)
                q1 = jnp.where(valid, area2 / (e1x * e1x + e1y * e1y), zerov)
                q2 = jnp.where(valid, area2 / (e2x * e2x + e2y * e2y), zerov)
                vals = (n0x, n0y, c0, n1x, n1y, c1, zx, zy, zc, q0, q1, q2)
                for k, v in enumerate(vals):
                    outv[pl.ds(k * per_w + j * _SC_L, _SC_L)] = v
            for k in range(_NCST):
                off = jax.lax.add(base, np.int32((b * _NCST + k) * fp3))
                pltpu.sync_copy(outv.at[pl.ds(k * per_w, per_w)],
                                out_h.at[pl.ds(off, per_w)])

    out = sck(vx, vy, vz, f0, f1, f2)
    return out.reshape(B, _NCST, fp3)


@jax.jit
def _run(verts, faces):
    B = verts.shape[0]
    F = faces.shape[0]
    H = W = IMAGE_SIZE
    faces_i = faces.astype(jnp.int32)
    fpad = ((F + _UNROLL - 1) // _UNROLL) * _UNROLL
    cst = _sc_face_constants(verts.astype(_F32), faces_i, fpad)
    fp3 = cst.shape[2]
    consts = [cst[:, k:k + 1, :] for k in range(_NCST)]

    z32 = lambda: jnp.int32(0)
    cspec = pl.BlockSpec((1, 1, fp3), lambda b, h: (b, z32(), z32()),
                         memory_space=pltpu.SMEM)
    ospec = pl.BlockSpec((1, _ROWS, W), lambda b, h: (b, h, z32()))
    outs = pl.pallas_call(
        functools.partial(_raster_kernel, F, fpad),
        grid=(B, H // _ROWS),
        in_specs=[cspec] * _NCST,
        out_specs=[ospec] * 6,
        out_shape=[
            jax.ShapeDtypeStruct((B, H, W), jnp.int32),
            jax.ShapeDtypeStruct((B, H, W), _F32),
            jax.ShapeDtypeStruct((B, H, W), _F32),
            jax.ShapeDtypeStruct((B, H, W), _F32),
            jax.ShapeDtypeStruct((B, H, W), _F32),
            jax.ShapeDtypeStruct((B, H, W), _F32),
        ],
    )(*consts)
    p2f_i, zb, b0, b1, b2, ds = outs
    pix_to_face = p2f_i.astype(jnp.int64)[..., None]
    zbuf = zb[..., None]
    bary = jnp.stack([b0, b1, b2], axis=-1)[:, :, :, None, :]
    dists = ds[..., None]
    return pix_to_face, zbuf, bary, dists


def kernel(verts, faces):
    return _run(verts, faces)


# speedup vs baseline: 1.3440x; 1.0912x over previous
import functools

import jax
import jax.numpy as jnp
import numpy as np
from jax.experimental import pallas as pl
from jax.experimental.pallas import tpu as pltpu
from jax.experimental.pallas import tpu_sc as plsc

IMAGE_SIZE = 128
EPS = 1e-8
_F32 = np.float32
_BIG = np.float32(1e30)
_TILES = 4
_ROWS = 8 * _TILES
_UNROLL = 8


def _raster_kernel(f_total, f_padded, *refs):
    (n0x_r, n0y_r, c0_r, n1x_r, n1y_r, c1_r,
     zx_r, zy_r, zc_r, q0_r, q1_r, q2_r,
     o_p2f, o_z, o_b0, o_b1, o_b2, o_d) = refs
    W = IMAGE_SIZE
    b = pl.program_id(0)
    hblk = pl.program_id(1)

    ix = jax.lax.broadcasted_iota(jnp.int32, (8, W), 1).astype(_F32)
    pxb = 1.0 - (2.0 * ix + 1.0) * _F32(1.0 / W)
    iy = jax.lax.broadcasted_iota(jnp.int32, (8, W), 0)
    base = hblk * _ROWS
    pybs = []
    for t in range(_TILES):
        yt = (base + t * 8 + iy).astype(_F32)
        pybs.append(_F32(1.0) - (2.0 * yt + 1.0) * _F32(1.0 / IMAGE_SIZE))

    zeros = jnp.zeros((8, W), _F32)

    st0 = []
    for t in range(_TILES):
        st0.extend([jnp.full((8, W), _BIG, _F32), zeros, zeros, zeros,
                    zeros, zeros, zeros])

    def body(i, st):
        del i
        st = list(st)
        fbase = st[-1]
        for k in range(_UNROLL):
            f = jax.lax.add(fbase, np.int32(k))
            n0x = n0x_r[0, 0, f]
            n0y = n0y_r[0, 0, f]
            c0 = c0_r[0, 0, f]
            n1x = n1x_r[0, 0, f]
            n1y = n1y_r[0, 0, f]
            c1 = c1_r[0, 0, f]
            zx = zx_r[0, 0, f]
            zy = zy_r[0, 0, f]
            zc = zc_r[0, 0, f]
            q0 = q0_r[0, 0, f]
            q1 = q1_r[0, 0, f]
            q2 = q2_r[0, 0, f]
            ff = f.astype(_F32)
            for t in range(_TILES):
                sti = 7 * t
                zrun, fidx, pb0, pb1, pq0, pq1, pq2 = st[sti: sti + 7]
                pyb = pybs[t]
                b0 = n0x * pxb + (n0y * pyb + c0)
                b1 = n1x * pxb + (n1y * pyb + c1)
                b2 = 1.0 - b0 - b1
                pz = zx * pxb + (zy * pyb + zc)
                m3 = jnp.minimum(jnp.minimum(b0, b1), b2)
                zcand = jnp.where(m3 >= 0.0, pz, _BIG)
                upd = zcand < zrun
                st[sti: sti + 7] = [
                    jnp.minimum(zcand, zrun),
                    jnp.where(upd, ff, fidx),
                    jnp.where(upd, b0, pb0),
                    jnp.where(upd, b1, pb1),
                    jnp.where(upd, q0, pq0),
                    jnp.where(upd, q1, pq1),
                    jnp.where(upd, q2, pq2),
                ]
        st[-1] = jax.lax.add(fbase, np.int32(_UNROLL))
        return tuple(st)

    st0.append(jnp.int32(0))
    st = jax.lax.fori_loop(np.int32(0), np.int32(f_padded // _UNROLL),
                           body, tuple(st0))

    bF = (b * f_total).astype(_F32)
    rows_p2f, rows_z, rows_b0, rows_b1, rows_b2, rows_d = [], [], [], [], [], []
    for t in range(_TILES):
        zrun, fidx, pb0, pb1, pq0, pq1, pq2 = st[7 * t: 7 * t + 7]
        pb2 = 1.0 - pb0 - pb1
        hit = zrun < _BIG
        d = jnp.minimum(jnp.minimum(pb0 * pb0 * pq0, pb1 * pb1 * pq1),
                        pb2 * pb2 * pq2)
        rows_p2f.append(jnp.where(hit, bF + fidx, _F32(-1.0)))
        rows_z.append(jnp.where(hit, zrun, _F32(-1.0)))
        rows_b0.append(jnp.where(hit, pb0, _F32(-1.0)))
        rows_b1.append(jnp.where(hit, pb1, _F32(-1.0)))
        rows_b2.append(jnp.where(hit, pb2, _F32(-1.0)))
        rows_d.append(jnp.where(hit, -d, _F32(-1.0)))
    o_p2f[0] = jnp.concatenate(rows_p2f, axis=0).astype(jnp.int32)
    o_z[0] = jnp.concatenate(rows_z, axis=0)
    o_b0[0] = jnp.concatenate(rows_b0, axis=0)
    o_b1[0] = jnp.concatenate(rows_b1, axis=0)
    o_b2[0] = jnp.concatenate(rows_b2, axis=0)
    o_d[0] = jnp.concatenate(rows_d, axis=0)


_SC_NC = 2
_SC_NS = 16
_SC_L = 16
_NCST = 12


def _sc_face_constants(verts, faces_i, fp):
    B, V, _ = verts.shape
    F = faces_i.shape[0]
    nw = _SC_NC * _SC_NS
    chunk = nw * _SC_L
    fp3 = ((max(F, fp) + chunk - 1) // chunk) * chunk
    per_w = fp3 // nw
    jn = per_w // _SC_L

    vx = verts[:, :, 0].reshape(B * V)
    vy = verts[:, :, 1].reshape(B * V)
    vz = verts[:, :, 2].reshape(B * V)
    f0 = jnp.pad(faces_i[:, 0], (0, fp3 - F))
    f1 = jnp.pad(faces_i[:, 1], (0, fp3 - F))
    f2 = jnp.pad(faces_i[:, 2], (0, fp3 - F))

    mesh = plsc.VectorSubcoreMesh(core_axis_name="c", subcore_axis_name="s")

    @functools.partial(
        pl.kernel, mesh=mesh,
        out_type=jax.ShapeDtypeStruct((B * _NCST * fp3,), jnp.float32),
        scratch_types=[
            pltpu.VMEM((per_w,), jnp.int32),
            pltpu.VMEM((per_w,), jnp.int32),
            pltpu.VMEM((per_w,), jnp.int32),
            pltpu.VMEM((per_w,), jnp.int32),
            pltpu.VMEM((per_w,), jnp.int32),
            pltpu.VMEM((per_w,), jnp.int32),
            pltpu.VMEM((9 * per_w,), jnp.float32),
            pltpu.VMEM((_NCST * per_w,), jnp.float32),
            pltpu.SemaphoreType.DMA,
        ],
    )
    def sck(vx_h, vy_h, vz_h, f0_h, f1_h, f2_h, out_h,
            f0v, f1v, f2v, i0v, i1v, i2v, gv, outv, sem):
        c = jax.lax.axis_index("c")
        s = jax.lax.axis_index("s")
        wid = jax.lax.add(jax.lax.mul(s, np.int32(_SC_NC)), c)
        base = jax.lax.mul(wid, np.int32(per_w))
        pltpu.sync_copy(f0_h.at[pl.ds(base, per_w)], f0v)
        pltpu.sync_copy(f1_h.at[pl.ds(base, per_w)], f1v)
        pltpu.sync_copy(f2_h.at[pl.ds(base, per_w)], f2v)
        onev = jnp.full((_SC_L,), _F32(1.0), jnp.float32)
        epsv = jnp.full((_SC_L,), _F32(EPS), jnp.float32)
        negv = jnp.full((_SC_L,), _F32(-1.0), jnp.float32)
        zerov = jnp.zeros((_SC_L,), jnp.float32)
        for b in range(B):
            boff = jnp.full((_SC_L,), b * V, jnp.int32)
            for j in range(jn):
                sl = pl.ds(j * _SC_L, _SC_L)
                i0v[sl] = f0v[sl] + boff
                i1v[sl] = f1v[sl] + boff
                i2v[sl] = f2v[sl] + boff
            copies = []
            for iv, row in ((i0v, 0), (i1v, 1), (i2v, 2)):
                for coord, src in enumerate((vx_h, vy_h, vz_h)):
                    dst = gv.at[pl.ds((row * 3 + coord) * per_w, per_w)]
                    copies.append(pltpu.async_copy(src.at[iv], dst, sem))
            for cp in copies:
                cp.wait()
            for j in range(jn):
                sl = pl.ds(j * _SC_L, _SC_L)
                def gld(row):
                    return gv[pl.ds(row * per_w + j * _SC_L, _SC_L)]

                x0 = gld(0)
                y0 = gld(1)
                z0 = gld(2)
                x1 = gld(3)
                y1 = gld(4)
                z1 = gld(5)
                x2 = gld(6)
                y2 = gld(7)
                z2 = gld(8)
                area = (x1 - x0) * (y2 - y0) - (y1 - y0) * (x2 - x0)
                absa = jnp.abs(area)
                valid = absa > epsv
                asafe = jnp.where(absa < epsv, epsv, area)
                inv = onev / asafe
                n0x = -(y2 - y1) * inv
                n0y = (x2 - x1) * inv
                c0 = ((y2 - y1) * x1 - (x2 - x1) * y1) * inv
                n1x = -(y0 - y2) * inv
                n1y = (x0 - x2) * inv
                c1 = ((y0 - y2) * x2 - (x0 - x2) * y2) * inv
                n2x = -(y1 - y0) * inv
                n2y = (x1 - x0) * inv
                c2 = ((y1 - y0) * x0 - (x1 - x0) * y0) * inv
                n0x = jnp.where(valid, n0x, zerov)
                n0y = jnp.where(valid, n0y, zerov)
                c0 = jnp.where(valid, c0, negv)
                zx = n0x * z0 + n1x * z1 + n2x * z2
                zy = n0y * z0 + n1y * z1 + n2y * z2
                zc = c0 * z0 + c1 * z1 + c2 * z2
                area2 = area * area
                e0x = x2 - x1
                e0y = y2 - y1
                e1x = x0 - x2
                e1y = y0 - y2
                e2x = x1 - x0
                e2y = y1 - y0
                q0 = jnp.where(valid, area2 / (e0x * e0x + e0y * e0y), zerov)
                q1 = jnp.where(valid, area2 / (e1x * e1x + e1y * e1y), zerov)
                q2 = jnp.where(valid, area2 / (e2x * e2x + e2y * e2y), zerov)
                vals = (n0x, n0y, c0, n1x, n1y, c1, zx, zy, zc, q0, q1, q2)
                for k, v in enumerate(vals):
                    outv[pl.ds(k * per_w + j * _SC_L, _SC_L)] = v
            for k in range(_NCST):
                off = jax.lax.add(base, np.int32((b * _NCST + k) * fp3))
                pltpu.sync_copy(outv.at[pl.ds(k * per_w, per_w)],
                                out_h.at[pl.ds(off, per_w)])

    out = sck(vx, vy, vz, f0, f1, f2)
    return out.reshape(B, _NCST, fp3)


@jax.jit
def _run(verts, faces):
    B = verts.shape[0]
    F = faces.shape[0]
    H = W = IMAGE_SIZE
    faces_i = faces.astype(jnp.int32)
    fpad = ((F + _UNROLL - 1) // _UNROLL) * _UNROLL
    cst = _sc_face_constants(verts.astype(_F32), faces_i, fpad)
    fp3 = cst.shape[2]
    consts = [cst[:, k:k + 1, :] for k in range(_NCST)]

    z32 = lambda: jnp.int32(0)
    cspec = pl.BlockSpec((1, 1, fp3), lambda b, h: (b, z32(), z32()),
                         memory_space=pltpu.SMEM)
    ospec = pl.BlockSpec((1, _ROWS, W), lambda b, h: (b, h, z32()))
    outs = pl.pallas_call(
        functools.partial(_raster_kernel, F, fpad),
        grid=(B, H // _ROWS),
        in_specs=[cspec] * _NCST,
        out_specs=[ospec] * 6,
        out_shape=[
            jax.ShapeDtypeStruct((B, H, W), jnp.int32),
            jax.ShapeDtypeStruct((B, H, W), _F32),
            jax.ShapeDtypeStruct((B, H, W), _F32),
            jax.ShapeDtypeStruct((B, H, W), _F32),
            jax.ShapeDtypeStruct((B, H, W), _F32),
            jax.ShapeDtypeStruct((B, H, W), _F32),
        ],
    )(*consts)
    p2f_i, zb, b0, b1, b2, ds = outs
    pix_to_face = p2f_i.astype(jnp.int64)[..., None]
    zbuf = zb[..., None]
    bary = jnp.stack([b0, b1, b2], axis=-1)[:, :, :, None, :]
    dists = ds[..., None]
    return pix_to_face, zbuf, bary, dists


def kernel(verts, faces):
    return _run(verts, faces)


# speedup vs baseline: 1.3593x; 1.0114x over previous
import functools

import jax
import jax.numpy as jnp
import numpy as np
from jax.experimental import pallas as pl
from jax.experimental.pallas import tpu as pltpu
from jax.experimental.pallas import tpu_sc as plsc

IMAGE_SIZE = 128
EPS = 1e-8
_F32 = np.float32
_BIG = np.float32(1e30)
_TILES = 4
_ROWS = 8 * _TILES
_UNROLL = 16


def _raster_kernel(f_total, f_padded, *refs):
    (n0x_r, n0y_r, c0_r, n1x_r, n1y_r, c1_r,
     zx_r, zy_r, zc_r, q0_r, q1_r, q2_r,
     o_p2f, o_z, o_b0, o_b1, o_b2, o_d) = refs
    W = IMAGE_SIZE
    b = pl.program_id(0)
    hblk = pl.program_id(1)

    ix = jax.lax.broadcasted_iota(jnp.int32, (8, W), 1).astype(_F32)
    pxb = 1.0 - (2.0 * ix + 1.0) * _F32(1.0 / W)
    iy = jax.lax.broadcasted_iota(jnp.int32, (8, W), 0)
    base = hblk * _ROWS
    pybs = []
    for t in range(_TILES):
        yt = (base + t * 8 + iy).astype(_F32)
        pybs.append(_F32(1.0) - (2.0 * yt + 1.0) * _F32(1.0 / IMAGE_SIZE))

    zeros = jnp.zeros((8, W), _F32)

    st0 = []
    for t in range(_TILES):
        st0.extend([jnp.full((8, W), _BIG, _F32), zeros, zeros, zeros,
                    zeros, zeros, zeros])

    def body(i, st):
        del i
        st = list(st)
        fbase = st[-1]
        for k in range(_UNROLL):
            f = jax.lax.add(fbase, np.int32(k))
            n0x = n0x_r[0, 0, f]
            n0y = n0y_r[0, 0, f]
            c0 = c0_r[0, 0, f]
            n1x = n1x_r[0, 0, f]
            n1y = n1y_r[0, 0, f]
            c1 = c1_r[0, 0, f]
            zx = zx_r[0, 0, f]
            zy = zy_r[0, 0, f]
            zc = zc_r[0, 0, f]
            q0 = q0_r[0, 0, f]
            q1 = q1_r[0, 0, f]
            q2 = q2_r[0, 0, f]
            ff = f.astype(_F32)
            for t in range(_TILES):
                sti = 7 * t
                zrun, fidx, pb0, pb1, pq0, pq1, pq2 = st[sti: sti + 7]
                pyb = pybs[t]
                b0 = n0x * pxb + (n0y * pyb + c0)
                b1 = n1x * pxb + (n1y * pyb + c1)
                b2 = 1.0 - b0 - b1
                pz = zx * pxb + (zy * pyb + zc)
                m3 = jnp.minimum(jnp.minimum(b0, b1), b2)
                zcand = jnp.where(m3 >= 0.0, pz, _BIG)
                upd = zcand < zrun
                st[sti: sti + 7] = [
                    jnp.minimum(zcand, zrun),
                    jnp.where(upd, ff, fidx),
                    jnp.where(upd, b0, pb0),
                    jnp.where(upd, b1, pb1),
                    jnp.where(upd, q0, pq0),
                    jnp.where(upd, q1, pq1),
                    jnp.where(upd, q2, pq2),
                ]
        st[-1] = jax.lax.add(fbase, np.int32(_UNROLL))
        return tuple(st)

    st0.append(jnp.int32(0))
    st = jax.lax.fori_loop(np.int32(0), np.int32(f_padded // _UNROLL),
                           body, tuple(st0))

    bF = (b * f_total).astype(_F32)
    rows_p2f, rows_z, rows_b0, rows_b1, rows_b2, rows_d = [], [], [], [], [], []
    for t in range(_TILES):
        zrun, fidx, pb0, pb1, pq0, pq1, pq2 = st[7 * t: 7 * t + 7]
        pb2 = 1.0 - pb0 - pb1
        hit = zrun < _BIG
        d = jnp.minimum(jnp.minimum(pb0 * pb0 * pq0, pb1 * pb1 * pq1),
                        pb2 * pb2 * pq2)
        rows_p2f.append(jnp.where(hit, bF + fidx, _F32(-1.0)))
        rows_z.append(jnp.where(hit, zrun, _F32(-1.0)))
        rows_b0.append(jnp.where(hit, pb0, _F32(-1.0)))
        rows_b1.append(jnp.where(hit, pb1, _F32(-1.0)))
        rows_b2.append(jnp.where(hit, pb2, _F32(-1.0)))
        rows_d.append(jnp.where(hit, -d, _F32(-1.0)))
    o_p2f[0] = jnp.concatenate(rows_p2f, axis=0).astype(jnp.int32)
    o_z[0] = jnp.concatenate(rows_z, axis=0)
    o_b0[0] = jnp.concatenate(rows_b0, axis=0)
    o_b1[0] = jnp.concatenate(rows_b1, axis=0)
    o_b2[0] = jnp.concatenate(rows_b2, axis=0)
    o_d[0] = jnp.concatenate(rows_d, axis=0)


_SC_NC = 2
_SC_NS = 16
_SC_L = 16
_NCST = 12


def _sc_face_constants(verts, faces_i, fp):
    B, V, _ = verts.shape
    F = faces_i.shape[0]
    nw = _SC_NC * _SC_NS
    chunk = nw * _SC_L
    fp3 = ((max(F, fp) + chunk - 1) // chunk) * chunk
    per_w = fp3 // nw
    jn = per_w // _SC_L

    vx = verts[:, :, 0].reshape(B * V)
    vy = verts[:, :, 1].reshape(B * V)
    vz = verts[:, :, 2].reshape(B * V)
    f0 = jnp.pad(faces_i[:, 0], (0, fp3 - F))
    f1 = jnp.pad(faces_i[:, 1], (0, fp3 - F))
    f2 = jnp.pad(faces_i[:, 2], (0, fp3 - F))

    mesh = plsc.VectorSubcoreMesh(core_axis_name="c", subcore_axis_name="s")

    @functools.partial(
        pl.kernel, mesh=mesh,
        out_type=jax.ShapeDtypeStruct((B * _NCST * fp3,), jnp.float32),
        scratch_types=[
            pltpu.VMEM((per_w,), jnp.int32),
            pltpu.VMEM((per_w,), jnp.int32),
            pltpu.VMEM((per_w,), jnp.int32),
            pltpu.VMEM((per_w,), jnp.int32),
            pltpu.VMEM((per_w,), jnp.int32),
            pltpu.VMEM((per_w,), jnp.int32),
            pltpu.VMEM((9 * per_w,), jnp.float32),
            pltpu.VMEM((_NCST * per_w,), jnp.float32),
            pltpu.SemaphoreType.DMA,
        ],
    )
    def sck(vx_h, vy_h, vz_h, f0_h, f1_h, f2_h, out_h,
            f0v, f1v, f2v, i0v, i1v, i2v, gv, outv, sem):
        c = jax.lax.axis_index("c")
        s = jax.lax.axis_index("s")
        wid = jax.lax.add(jax.lax.mul(s, np.int32(_SC_NC)), c)
        base = jax.lax.mul(wid, np.int32(per_w))
        pltpu.sync_copy(f0_h.at[pl.ds(base, per_w)], f0v)
        pltpu.sync_copy(f1_h.at[pl.ds(base, per_w)], f1v)
        pltpu.sync_copy(f2_h.at[pl.ds(base, per_w)], f2v)
        onev = jnp.full((_SC_L,), _F32(1.0), jnp.float32)
        epsv = jnp.full((_SC_L,), _F32(EPS), jnp.float32)
        negv = jnp.full((_SC_L,), _F32(-1.0), jnp.float32)
        zerov = jnp.zeros((_SC_L,), jnp.float32)
        for b in range(B):
            boff = jnp.full((_SC_L,), b * V, jnp.int32)
            for j in range(jn):
                sl = pl.ds(j * _SC_L, _SC_L)
                i0v[sl] = f0v[sl] + boff
                i1v[sl] = f1v[sl] + boff
                i2v[sl] = f2v[sl] + boff
            copies = []
            for iv, row in ((i0v, 0), (i1v, 1), (i2v, 2)):
                for coord, src in enumerate((vx_h, vy_h, vz_h)):
                    dst = gv.at[pl.ds((row * 3 + coord) * per_w, per_w)]
                    copies.append(pltpu.async_copy(src.at[iv], dst, sem))
            for cp in copies:
                cp.wait()
            for j in range(jn):
                sl = pl.ds(j * _SC_L, _SC_L)
                def gld(row):
                    return gv[pl.ds(row * per_w + j * _SC_L, _SC_L)]

                x0 = gld(0)
                y0 = gld(1)
                z0 = gld(2)
                x1 = gld(3)
                y1 = gld(4)
                z1 = gld(5)
                x2 = gld(6)
                y2 = gld(7)
                z2 = gld(8)
                area = (x1 - x0) * (y2 - y0) - (y1 - y0) * (x2 - x0)
                absa = jnp.abs(area)
                valid = absa > epsv
                asafe = jnp.where(absa < epsv, epsv, area)
                inv = onev / asafe
                n0x = -(y2 - y1) * inv
                n0y = (x2 - x1) * inv
                c0 = ((y2 - y1) * x1 - (x2 - x1) * y1) * inv
                n1x = -(y0 - y2) * inv
                n1y = (x0 - x2) * inv
                c1 = ((y0 - y2) * x2 - (x0 - x2) * y2) * inv
                n2x = -(y1 - y0) * inv
                n2y = (x1 - x0) * inv
                c2 = ((y1 - y0) * x0 - (x1 - x0) * y0) * inv
                n0x = jnp.where(valid, n0x, zerov)
                n0y = jnp.where(valid, n0y, zerov)
                c0 = jnp.where(valid, c0, negv)
                zx = n0x * z0 + n1x * z1 + n2x * z2
                zy = n0y * z0 + n1y * z1 + n2y * z2
                zc = c0 * z0 + c1 * z1 + c2 * z2
                area2 = area * area
                e0x = x2 - x1
                e0y = y2 - y1
                e1x = x0 - x2
                e1y = y0 - y2
                e2x = x1 - x0
                e2y = y1 - y0
                q0 = jnp.where(valid, area2 / (e0x * e0x + e0y * e0y), zerov)
                q1 = jnp.where(valid, area2 / (e1x * e1x + e1y * e1y), zerov)
                q2 = jnp.where(valid, area2 / (e2x * e2x + e2y * e2y), zerov)
                vals = (n0x, n0y, c0, n1x, n1y, c1, zx, zy, zc, q0, q1, q2)
                for k, v in enumerate(vals):
                    outv[pl.ds(k * per_w + j * _SC_L, _SC_L)] = v
            for k in range(_NCST):
                off = jax.lax.add(base, np.int32((b * _NCST + k) * fp3))
                pltpu.sync_copy(outv.at[pl.ds(k * per_w, per_w)],
                                out_h.at[pl.ds(off, per_w)])

    out = sck(vx, vy, vz, f0, f1, f2)
    return out.reshape(B, _NCST, fp3)


@jax.jit
def _run(verts, faces):
    B = verts.shape[0]
    F = faces.shape[0]
    H = W = IMAGE_SIZE
    faces_i = faces.astype(jnp.int32)
    fpad = ((F + _UNROLL - 1) // _UNROLL) * _UNROLL
    cst = _sc_face_constants(verts.astype(_F32), faces_i, fpad)
    fp3 = cst.shape[2]
    consts = [cst[:, k:k + 1, :] for k in range(_NCST)]

    z32 = lambda: jnp.int32(0)
    cspec = pl.BlockSpec((1, 1, fp3), lambda b, h: (b, z32(), z32()),
                         memory_space=pltpu.SMEM)
    ospec = pl.BlockSpec((1, _ROWS, W), lambda b, h: (b, h, z32()))
    outs = pl.pallas_call(
        functools.partial(_raster_kernel, F, fpad),
        grid=(B, H // _ROWS),
        in_specs=[cspec] * _NCST,
        out_specs=[ospec] * 6,
        out_shape=[
            jax.ShapeDtypeStruct((B, H, W), jnp.int32),
            jax.ShapeDtypeStruct((B, H, W), _F32),
            jax.ShapeDtypeStruct((B, H, W), _F32),
            jax.ShapeDtypeStruct((B, H, W), _F32),
            jax.ShapeDtypeStruct((B, H, W), _F32),
            jax.ShapeDtypeStruct((B, H, W), _F32),
        ],
    )(*consts)
    p2f_i, zb, b0, b1, b2, ds = outs
    pix_to_face = p2f_i.astype(jnp.int64)[..., None]
    zbuf = zb[..., None]
    bary = jnp.stack([b0, b1, b2], axis=-1)[:, :, :, None, :]
    dists = ds[..., None]
    return pix_to_face, zbuf, bary, dists


def kernel(verts, faces):
    return _run(verts, faces)
